# baseline (speedup 1.0000x reference)
import functools

import jax
import jax.numpy as jnp
from jax import lax
from jax.experimental import pallas as pl
from jax.experimental.pallas import tpu as pltpu
from jax.experimental.pallas import tpu_sc as plsc

MEM = 2048
DIM = 1024
N = 4096
RB = 512
NBLK = 2048 // RB


def _stats_body(att_ref, colsum_ref, ent_ref, colacc_ref):
    rb = pl.program_id(1)
    a = att_ref[0]

    acc = a[0:8, :]
    for t in range(1, RB // 8):
        acc = acc + a[t * 8:(t + 1) * 8, :]
    q = acc[0:4, :] + acc[4:8, :]
    r = q[0:2, :] + q[2:4, :]
    bsum = r[0:1, :] + r[1:2, :]

    @pl.when(rb == 0)
    def _():
        colacc_ref[...] = bsum

    @pl.when(rb > 0)
    def _():
        colacc_ref[...] = colacc_ref[...] + bsum

    @pl.when(rb == NBLK - 1)
    def _():
        colsum_ref[0] = colacc_ref[...]

    z = jnp.sum(a, axis=1, keepdims=True)
    p = a / (z + 1e-10)
    ent_ref[...] = -jnp.sum(p * jnp.log(p + 1e-10), axis=1, keepdims=True)


_stats_call = pl.pallas_call(
    _stats_body,
    grid=(2, NBLK),
    in_specs=[pl.BlockSpec((1, RB, 2048), lambda b, rb: (b, rb, 0))],
    out_specs=[
        pl.BlockSpec((1, 1, 2048), lambda b, rb: (b, 0, 0)),
        pl.BlockSpec((RB, 1), lambda b, rb: (b * NBLK + rb, 0)),
    ],
    out_shape=[
        jax.ShapeDtypeStruct((2, 1, 2048), jnp.float32),
        jax.ShapeDtypeStruct((N, 1), jnp.float32),
    ],
    scratch_shapes=[pltpu.VMEM((1, 2048), jnp.float32)],
)


CH = 256


def _rank_body(imp_row_ref, ent_ref,
               active_ref, wsrc_ref, wslot_ref, em_ref, ev_ref):
    vrow = imp_row_ref[...]
    vcol = jnp.reshape(vrow, (N, 1))

    jj = lax.broadcasted_iota(jnp.int32, (N, CH), 0)
    rank_chunks = []
    for c in range(N // CH):
        vi = vrow[0:1, c * CH:(c + 1) * CH]
        ii = c * CH + lax.broadcasted_iota(jnp.int32, (N, CH), 1)
        win = (vcol > vi) | ((vcol == vi) & (jj < ii))
        rank_chunks.append(jnp.sum(win.astype(jnp.float32), axis=0,
                                   keepdims=True))
    rank = jnp.concatenate(rank_chunks, axis=1)
    rank_col = jnp.reshape(rank, (N, 1))

    jrow = lax.broadcasted_iota(jnp.int32, (1, N), 1).astype(jnp.float32)
    act_chunks = []
    for c in range(MEM // CH):
        r_row = jnp.float32(c * CH) + lax.broadcasted_iota(
            jnp.int32, (1, CH), 1).astype(jnp.float32)
        eq = (rank_col == r_row).astype(jnp.float32)
        act_chunks.append(jnp.dot(jrow, eq, precision=lax.Precision.HIGHEST,
                                  preferred_element_type=jnp.float32))
    active_ref[...] = jnp.concatenate(act_chunks, axis=1).astype(jnp.int32)

    r1 = rank[0:1, 0:MEM]
    r2 = rank[0:1, MEM:N]
    kf = jnp.float32(MEM)
    sel1 = r1 < kf
    sel2 = r2 < kf
    flag = sel1 | sel2
    s_iota = lax.broadcasted_iota(jnp.int32, (1, MEM), 1).astype(jnp.float32)
    src = jnp.where(sel1 & sel2,
                    jnp.where(r1 > r2, s_iota, s_iota + kf),
                    jnp.where(sel1, s_iota, s_iota + kf))

    flagf = flag.astype(jnp.float32)
    incl = flagf
    sh = 1
    lane = lax.broadcasted_iota(jnp.int32, (1, MEM), 1)
    while sh < MEM:
        rolled = pltpu.roll(incl, sh, 1)
        incl = incl + jnp.where(lane >= sh, rolled, 0.0)
        sh *= 2
    pos = jnp.where(flag, incl - flagf, -1.0)
    n1 = jnp.sum(flagf)

    pos_col = jnp.reshape(pos, (MEM, 1))
    wsrc_chunks = []
    wslot_chunks = []
    for c in range(MEM // CH):
        r_row = jnp.float32(c * CH) + lax.broadcasted_iota(
            jnp.int32, (1, CH), 1).astype(jnp.float32)
        rr = jnp.where(r_row < n1, r_row, r_row - n1)
        eq = (pos_col == rr).astype(jnp.float32)
        wsrc_chunks.append(jnp.dot(src, eq, precision=lax.Precision.HIGHEST,
                                   preferred_element_type=jnp.float32))
        wslot_chunks.append(jnp.dot(s_iota, eq, precision=lax.Precision.HIGHEST,
                                    preferred_element_type=jnp.float32))
    wsrc_ref[...] = jnp.concatenate(wsrc_chunks, axis=1).astype(jnp.int32)
    wslot_ref[...] = jnp.concatenate(wslot_chunks, axis=1).astype(jnp.int32)

    ent = ent_ref[...]
    m = jnp.sum(ent) / jnp.float32(N)
    d = ent - m
    em_ref[...] = jnp.reshape(m, (1, 1))
    ev_ref[...] = jnp.reshape(jnp.sum(d * d) / jnp.float32(N), (1, 1))


_rank_call = pl.pallas_call(
    _rank_body,
    out_shape=[
        jax.ShapeDtypeStruct((1, MEM), jnp.int32),
        jax.ShapeDtypeStruct((1, MEM), jnp.int32),
        jax.ShapeDtypeStruct((1, MEM), jnp.int32),
        jax.ShapeDtypeStruct((1, 1), jnp.float32),
        jax.ShapeDtypeStruct((1, 1), jnp.float32),
    ],
)


_ROWS_PER_W = MEM // 32


@functools.cache
def _sc_kernels():
    mesh = plsc.VectorSubcoreMesh(core_axis_name="c", subcore_axis_name="s")

    @functools.partial(
        pl.kernel,
        mesh=mesh,
        out_type=(
            jax.ShapeDtypeStruct((MEM, DIM), jnp.float32),
            jax.ShapeDtypeStruct((MEM, DIM), jnp.float32),
        ),
        scratch_types=[pltpu.VMEM((_ROWS_PER_W, DIM), jnp.float32)],
    )
    def sc_copy(mk_hbm, mv_hbm, ok_hbm, ov_hbm, buf):
        wid = lax.axis_index("s") * 2 + lax.axis_index("c")
        base = wid * _ROWS_PER_W
        pltpu.sync_copy(mk_hbm.at[pl.ds(base, _ROWS_PER_W)], buf)
        pltpu.sync_copy(buf, ok_hbm.at[pl.ds(base, _ROWS_PER_W)])
        pltpu.sync_copy(mv_hbm.at[pl.ds(base, _ROWS_PER_W)], buf)
        pltpu.sync_copy(buf, ov_hbm.at[pl.ds(base, _ROWS_PER_W)])

    _NCH = 4
    _CW = _ROWS_PER_W // _NCH

    @functools.partial(
        pl.kernel,
        mesh=mesh,
        out_type=(),
        scratch_types=[
            pltpu.VMEM((_NCH, _CW), jnp.int32),
            pltpu.VMEM((_NCH, _CW), jnp.int32),
            pltpu.VMEM((_NCH, _CW, DIM), jnp.float32),
            pltpu.SemaphoreType.DMA,
            pltpu.SemaphoreType.DMA,
        ],
    )
    def sc_scatter(hid_hbm, wsrc_hbm, wslot_hbm, ok_ref, ov_ref,
                   srcbuf, slotbuf, rowbuf, gsem, ssem):
        wid = lax.axis_index("s") * 2 + lax.axis_index("c")
        pltpu.sync_copy(wsrc_hbm.at[wid], srcbuf)
        pltpu.sync_copy(wslot_hbm.at[wid], slotbuf)
        gathers = [
            pltpu.async_copy(hid_hbm.at[srcbuf.at[c]], rowbuf.at[c], gsem)
            for c in range(_NCH)
        ]
        scats = []
        for c in range(_NCH):
            gathers[c].wait()
            scats.append(
                pltpu.async_copy(rowbuf.at[c], ok_ref.at[slotbuf.at[c]], ssem))
            scats.append(
                pltpu.async_copy(rowbuf.at[c], ov_ref.at[slotbuf.at[c]], ssem))
        for s in scats:
            s.wait()

    return sc_copy, sc_scatter


def kernel(hidden_states, attention_weights, memory_keys, memory_values):
    B, S, D = hidden_states.shape
    hid = hidden_states.reshape(B * S, D)

    sc_copy, sc_scatter = _sc_kernels()
    base_k, base_v = sc_copy(memory_keys, memory_values)

    colsum3, ent_col = _stats_call(attention_weights)
    imp_row = colsum3.reshape(1, N)

    active_row, wsrc_row, wslot_row, em, ev = _rank_call(imp_row, ent_col)

    kref = jax.new_ref(base_k)
    vref = jax.new_ref(base_v)
    sc_scatter(hid, wsrc_row.reshape(32, 4, _ROWS_PER_W // 4),
               wslot_row.reshape(32, 4, _ROWS_PER_W // 4), kref, vref)

    sparse_k = kref[...]
    sparse_v = vref[...]
    return (sparse_k, sparse_v, active_row.reshape(MEM),
            em.reshape(()), ev.reshape(()))


# speedup vs baseline: 1.3020x; 1.3020x over previous
import functools

import jax
import jax.numpy as jnp
from jax import lax
from jax.experimental import pallas as pl
from jax.experimental.pallas import tpu as pltpu
from jax.experimental.pallas import tpu_sc as plsc

MEM = 2048
DIM = 1024
N = 4096
RB = 512
NBLK = 2048 // RB


def _stats_body(att_ref, colsum_ref, ent_ref, colacc_ref):
    rb = pl.program_id(1)
    a = att_ref[0]

    acc = a[0:8, :]
    for t in range(1, RB // 8):
        acc = acc + a[t * 8:(t + 1) * 8, :]
    q = acc[0:4, :] + acc[4:8, :]
    r = q[0:2, :] + q[2:4, :]
    bsum = r[0:1, :] + r[1:2, :]

    @pl.when(rb == 0)
    def _():
        colacc_ref[...] = bsum

    @pl.when(rb > 0)
    def _():
        colacc_ref[...] = colacc_ref[...] + bsum

    @pl.when(rb == NBLK - 1)
    def _():
        colsum_ref[0] = colacc_ref[...]

    z = jnp.sum(a, axis=1, keepdims=True)
    p = a / (z + 1e-10)
    ent_ref[...] = -jnp.sum(p * jnp.log(p + 1e-10), axis=1, keepdims=True)


_stats_call = pl.pallas_call(
    _stats_body,
    grid=(2, NBLK),
    in_specs=[pl.BlockSpec((1, RB, 2048), lambda b, rb: (b, rb, 0))],
    out_specs=[
        pl.BlockSpec((1, 1, 2048), lambda b, rb: (b, 0, 0)),
        pl.BlockSpec((RB, 1), lambda b, rb: (b * NBLK + rb, 0)),
    ],
    out_shape=[
        jax.ShapeDtypeStruct((2, 1, 2048), jnp.float32),
        jax.ShapeDtypeStruct((N, 1), jnp.float32),
    ],
    scratch_shapes=[pltpu.VMEM((1, 2048), jnp.float32)],
)


CH = 256


def _rank_body(imp_row_ref, ent_ref,
               active_ref, wsrc_ref, wslot_ref, em_ref, ev_ref):
    vrow = imp_row_ref[...]
    vcol = jnp.reshape(vrow, (N, 1))

    jj = lax.broadcasted_iota(jnp.int32, (N, CH), 0)
    rank_chunks = []
    for c in range(N // CH):
        vi = vrow[0:1, c * CH:(c + 1) * CH]
        ii = c * CH + lax.broadcasted_iota(jnp.int32, (N, CH), 1)
        win = (vcol > vi) | ((vcol == vi) & (jj < ii))
        rank_chunks.append(jnp.sum(win.astype(jnp.float32), axis=0,
                                   keepdims=True))
    rank = jnp.concatenate(rank_chunks, axis=1)
    rank_col = jnp.reshape(rank, (N, 1))

    jcol = lax.broadcasted_iota(jnp.int32, (N, 1), 0).astype(jnp.float32)
    act_chunks = []
    for c in range(MEM // CH):
        r_row = jnp.float32(c * CH) + lax.broadcasted_iota(
            jnp.int32, (1, CH), 1).astype(jnp.float32)
        eq = (rank_col == r_row).astype(jnp.float32)
        act_chunks.append(jnp.sum(jcol * eq, axis=0, keepdims=True))
    active_ref[...] = jnp.concatenate(act_chunks, axis=1).astype(jnp.int32)

    r1 = rank[0:1, 0:MEM]
    r2 = rank[0:1, MEM:N]
    kf = jnp.float32(MEM)
    sel1 = r1 < kf
    sel2 = r2 < kf
    flag = sel1 | sel2
    s_iota = lax.broadcasted_iota(jnp.int32, (1, MEM), 1).astype(jnp.float32)
    src = jnp.where(sel1 & sel2,
                    jnp.where(r1 > r2, s_iota, s_iota + kf),
                    jnp.where(sel1, s_iota, s_iota + kf))

    flagf = flag.astype(jnp.float32)
    incl = flagf
    sh = 1
    lane = lax.broadcasted_iota(jnp.int32, (1, MEM), 1)
    while sh < MEM:
        rolled = pltpu.roll(incl, sh, 1)
        incl = incl + jnp.where(lane >= sh, rolled, 0.0)
        sh *= 2
    pos = jnp.where(flag, incl - flagf, -1.0)
    n1 = jnp.sum(flagf)

    pos_col = jnp.reshape(pos, (MEM, 1))
    src_col = jnp.reshape(src, (MEM, 1))
    s_col = lax.broadcasted_iota(jnp.int32, (MEM, 1), 0).astype(jnp.float32)
    wsrc_chunks = []
    wslot_chunks = []
    for c in range(MEM // CH):
        r_row = jnp.float32(c * CH) + lax.broadcasted_iota(
            jnp.int32, (1, CH), 1).astype(jnp.float32)
        rr = jnp.where(r_row < n1, r_row, r_row - n1)
        eq = (pos_col == rr).astype(jnp.float32)
        wsrc_chunks.append(jnp.sum(src_col * eq, axis=0, keepdims=True))
        wslot_chunks.append(jnp.sum(s_col * eq, axis=0, keepdims=True))
    wsrc_ref[...] = jnp.concatenate(wsrc_chunks, axis=1).astype(jnp.int32)
    wslot_ref[...] = jnp.concatenate(wslot_chunks, axis=1).astype(jnp.int32)

    ent = ent_ref[...]
    m = jnp.sum(ent) / jnp.float32(N)
    d = ent - m
    em_ref[...] = jnp.reshape(m, (1, 1))
    ev_ref[...] = jnp.reshape(jnp.sum(d * d) / jnp.float32(N), (1, 1))


_rank_call = pl.pallas_call(
    _rank_body,
    out_shape=[
        jax.ShapeDtypeStruct((1, MEM), jnp.int32),
        jax.ShapeDtypeStruct((1, MEM), jnp.int32),
        jax.ShapeDtypeStruct((1, MEM), jnp.int32),
        jax.ShapeDtypeStruct((1, 1), jnp.float32),
        jax.ShapeDtypeStruct((1, 1), jnp.float32),
    ],
)


_ROWS_PER_W = MEM // 32


@functools.cache
def _sc_kernels():
    mesh = plsc.VectorSubcoreMesh(core_axis_name="c", subcore_axis_name="s")

    @functools.partial(
        pl.kernel,
        mesh=mesh,
        out_type=(
            jax.ShapeDtypeStruct((MEM, DIM), jnp.float32),
            jax.ShapeDtypeStruct((MEM, DIM), jnp.float32),
        ),
        scratch_types=[pltpu.VMEM((_ROWS_PER_W, DIM), jnp.float32)],
    )
    def sc_copy(mk_hbm, mv_hbm, ok_hbm, ov_hbm, buf):
        wid = lax.axis_index("s") * 2 + lax.axis_index("c")
        base = wid * _ROWS_PER_W
        pltpu.sync_copy(mk_hbm.at[pl.ds(base, _ROWS_PER_W)], buf)
        pltpu.sync_copy(buf, ok_hbm.at[pl.ds(base, _ROWS_PER_W)])
        pltpu.sync_copy(mv_hbm.at[pl.ds(base, _ROWS_PER_W)], buf)
        pltpu.sync_copy(buf, ov_hbm.at[pl.ds(base, _ROWS_PER_W)])

    _NCH = 4
    _CW = _ROWS_PER_W // _NCH

    @functools.partial(
        pl.kernel,
        mesh=mesh,
        out_type=(),
        scratch_types=[
            pltpu.VMEM((_NCH, _CW), jnp.int32),
            pltpu.VMEM((_NCH, _CW), jnp.int32),
            pltpu.VMEM((_NCH, _CW, DIM), jnp.float32),
            pltpu.SemaphoreType.DMA,
            pltpu.SemaphoreType.DMA,
        ],
    )
    def sc_scatter(hid_hbm, wsrc_hbm, wslot_hbm, ok_ref, ov_ref,
                   srcbuf, slotbuf, rowbuf, gsem, ssem):
        wid = lax.axis_index("s") * 2 + lax.axis_index("c")
        pltpu.sync_copy(wsrc_hbm.at[wid], srcbuf)
        pltpu.sync_copy(wslot_hbm.at[wid], slotbuf)
        gathers = [
            pltpu.async_copy(hid_hbm.at[srcbuf.at[c]], rowbuf.at[c], gsem)
            for c in range(_NCH)
        ]
        scats = []
        for c in range(_NCH):
            gathers[c].wait()
            scats.append(
                pltpu.async_copy(rowbuf.at[c], ok_ref.at[slotbuf.at[c]], ssem))
            scats.append(
                pltpu.async_copy(rowbuf.at[c], ov_ref.at[slotbuf.at[c]], ssem))
        for s in scats:
            s.wait()

    return sc_copy, sc_scatter


def kernel(hidden_states, attention_weights, memory_keys, memory_values):
    B, S, D = hidden_states.shape
    hid = hidden_states.reshape(B * S, D)

    sc_copy, sc_scatter = _sc_kernels()
    base_k, base_v = sc_copy(memory_keys, memory_values)

    colsum3, ent_col = _stats_call(attention_weights)
    imp_row = colsum3.reshape(1, N)

    active_row, wsrc_row, wslot_row, em, ev = _rank_call(imp_row, ent_col)

    kref = jax.new_ref(base_k)
    vref = jax.new_ref(base_v)
    sc_scatter(hid, wsrc_row.reshape(32, 4, _ROWS_PER_W // 4),
               wslot_row.reshape(32, 4, _ROWS_PER_W // 4), kref, vref)

    sparse_k = kref[...]
    sparse_v = vref[...]
    return (sparse_k, sparse_v, active_row.reshape(MEM),
            em.reshape(()), ev.reshape(()))


# speedup vs baseline: 1.3278x; 1.0198x over previous
import functools

import jax
import jax.numpy as jnp
from jax import lax
from jax.experimental import pallas as pl
from jax.experimental.pallas import tpu as pltpu
from jax.experimental.pallas import tpu_sc as plsc

MEM = 2048
DIM = 1024
N = 4096
RB = 512
NBLK = 2048 // RB


def _stats_body(att_ref, colsum_ref, ent_ref, colacc_ref):
    rb = pl.program_id(1)
    a = att_ref[0]

    acc = a[0:8, :]
    for t in range(1, RB // 8):
        acc = acc + a[t * 8:(t + 1) * 8, :]
    q = acc[0:4, :] + acc[4:8, :]
    r = q[0:2, :] + q[2:4, :]
    bsum = r[0:1, :] + r[1:2, :]

    @pl.when(rb == 0)
    def _():
        colacc_ref[...] = bsum

    @pl.when(rb > 0)
    def _():
        colacc_ref[...] = colacc_ref[...] + bsum

    @pl.when(rb == NBLK - 1)
    def _():
        colsum_ref[0] = colacc_ref[...]

    z = jnp.sum(a, axis=1, keepdims=True)
    p = a / (z + 1e-10)
    ent_ref[...] = -jnp.sum(p * jnp.log(p + 1e-10), axis=1, keepdims=True)


_stats_call = pl.pallas_call(
    _stats_body,
    grid=(2, NBLK),
    in_specs=[pl.BlockSpec((1, RB, 2048), lambda b, rb: (b, rb, 0))],
    out_specs=[
        pl.BlockSpec((1, 1, 2048), lambda b, rb: (b, 0, 0)),
        pl.BlockSpec((RB, 1), lambda b, rb: (b * NBLK + rb, 0)),
    ],
    out_shape=[
        jax.ShapeDtypeStruct((2, 1, 2048), jnp.float32),
        jax.ShapeDtypeStruct((N, 1), jnp.float32),
    ],
    scratch_shapes=[pltpu.VMEM((1, 2048), jnp.float32)],
)


CH = 256


def _rank_body(imp_row_ref, ent_ref,
               active_ref, wsrc_ref, wslot_ref, em_ref, ev_ref):
    vrow = imp_row_ref[...]
    vcol = jnp.reshape(vrow, (N, 1))

    nb = N // CH
    jj = lax.broadcasted_iota(jnp.int32, (CH, CH), 0)
    ii = lax.broadcasted_iota(jnp.int32, (CH, CH), 1)
    vi_rows = [vrow[0:1, b * CH:(b + 1) * CH] for b in range(nb)]
    vj_cols = [vcol[b * CH:(b + 1) * CH] for b in range(nb)]
    acc = []
    for b in range(nb):
        win = ((vj_cols[b] > vi_rows[b])
               | ((vj_cols[b] == vi_rows[b]) & (jj < ii)))
        acc.append(jnp.sum(win.astype(jnp.float32), axis=0, keepdims=True))
    for bi in range(nb):
        for bj in range(bi + 1, nb):
            c = (vj_cols[bj] > vi_rows[bi]).astype(jnp.float32)
            acc[bi] = acc[bi] + jnp.sum(c, axis=0, keepdims=True)
            sj = jnp.sum(c, axis=1, keepdims=True)
            acc[bj] = acc[bj] + (jnp.float32(CH) - jnp.reshape(sj, (1, CH)))
    rank = jnp.concatenate(acc, axis=1)
    rank_col = jnp.reshape(rank, (N, 1))

    jcol = lax.broadcasted_iota(jnp.int32, (N, 1), 0).astype(jnp.float32)
    act_chunks = []
    for c in range(MEM // CH):
        r_row = jnp.float32(c * CH) + lax.broadcasted_iota(
            jnp.int32, (1, CH), 1).astype(jnp.float32)
        eq = (rank_col == r_row).astype(jnp.float32)
        act_chunks.append(jnp.sum(jcol * eq, axis=0, keepdims=True))
    active_ref[...] = jnp.concatenate(act_chunks, axis=1).astype(jnp.int32)

    r1 = rank[0:1, 0:MEM]
    r2 = rank[0:1, MEM:N]
    kf = jnp.float32(MEM)
    sel1 = r1 < kf
    sel2 = r2 < kf
    flag = sel1 | sel2
    s_iota = lax.broadcasted_iota(jnp.int32, (1, MEM), 1).astype(jnp.float32)
    src = jnp.where(sel1 & sel2,
                    jnp.where(r1 > r2, s_iota, s_iota + kf),
                    jnp.where(sel1, s_iota, s_iota + kf))

    flagf = flag.astype(jnp.float32)
    incl = flagf
    sh = 1
    lane = lax.broadcasted_iota(jnp.int32, (1, MEM), 1)
    while sh < MEM:
        rolled = pltpu.roll(incl, sh, 1)
        incl = incl + jnp.where(lane >= sh, rolled, 0.0)
        sh *= 2
    pos = jnp.where(flag, incl - flagf, -1.0)
    n1 = jnp.sum(flagf)

    pos_col = jnp.reshape(pos, (MEM, 1))
    src_col = jnp.reshape(src, (MEM, 1))
    s_col = lax.broadcasted_iota(jnp.int32, (MEM, 1), 0).astype(jnp.float32)
    wsrc_chunks = []
    wslot_chunks = []
    for c in range(MEM // CH):
        r_row = jnp.float32(c * CH) + lax.broadcasted_iota(
            jnp.int32, (1, CH), 1).astype(jnp.float32)
        rr = jnp.where(r_row < n1, r_row, r_row - n1)
        eq = (pos_col == rr).astype(jnp.float32)
        wsrc_chunks.append(jnp.sum(src_col * eq, axis=0, keepdims=True))
        wslot_chunks.append(jnp.sum(s_col * eq, axis=0, keepdims=True))
    wsrc_ref[...] = jnp.concatenate(wsrc_chunks, axis=1).astype(jnp.int32)
    wslot_ref[...] = jnp.concatenate(wslot_chunks, axis=1).astype(jnp.int32)

    ent = ent_ref[...]
    m = jnp.sum(ent) / jnp.float32(N)
    d = ent - m
    em_ref[...] = jnp.reshape(m, (1, 1))
    ev_ref[...] = jnp.reshape(jnp.sum(d * d) / jnp.float32(N), (1, 1))


_rank_call = pl.pallas_call(
    _rank_body,
    out_shape=[
        jax.ShapeDtypeStruct((1, MEM), jnp.int32),
        jax.ShapeDtypeStruct((1, MEM), jnp.int32),
        jax.ShapeDtypeStruct((1, MEM), jnp.int32),
        jax.ShapeDtypeStruct((1, 1), jnp.float32),
        jax.ShapeDtypeStruct((1, 1), jnp.float32),
    ],
)


_ROWS_PER_W = MEM // 32


@functools.cache
def _sc_kernels():
    mesh = plsc.VectorSubcoreMesh(core_axis_name="c", subcore_axis_name="s")

    @functools.partial(
        pl.kernel,
        mesh=mesh,
        out_type=(
            jax.ShapeDtypeStruct((MEM, DIM), jnp.float32),
            jax.ShapeDtypeStruct((MEM, DIM), jnp.float32),
        ),
        scratch_types=[pltpu.VMEM((_ROWS_PER_W, DIM), jnp.float32)],
    )
    def sc_copy(mk_hbm, mv_hbm, ok_hbm, ov_hbm, buf):
        wid = lax.axis_index("s") * 2 + lax.axis_index("c")
        base = wid * _ROWS_PER_W
        pltpu.sync_copy(mk_hbm.at[pl.ds(base, _ROWS_PER_W)], buf)
        pltpu.sync_copy(buf, ok_hbm.at[pl.ds(base, _ROWS_PER_W)])
        pltpu.sync_copy(mv_hbm.at[pl.ds(base, _ROWS_PER_W)], buf)
        pltpu.sync_copy(buf, ov_hbm.at[pl.ds(base, _ROWS_PER_W)])

    _NCH = 4
    _CW = _ROWS_PER_W // _NCH

    @functools.partial(
        pl.kernel,
        mesh=mesh,
        out_type=(),
        scratch_types=[
            pltpu.VMEM((_NCH, _CW), jnp.int32),
            pltpu.VMEM((_NCH, _CW), jnp.int32),
            pltpu.VMEM((_NCH, _CW, DIM), jnp.float32),
            pltpu.SemaphoreType.DMA,
            pltpu.SemaphoreType.DMA,
        ],
    )
    def sc_scatter(hid_hbm, wsrc_hbm, wslot_hbm, ok_ref, ov_ref,
                   srcbuf, slotbuf, rowbuf, gsem, ssem):
        wid = lax.axis_index("s") * 2 + lax.axis_index("c")
        pltpu.sync_copy(wsrc_hbm.at[wid], srcbuf)
        pltpu.sync_copy(wslot_hbm.at[wid], slotbuf)
        gathers = [
            pltpu.async_copy(hid_hbm.at[srcbuf.at[c]], rowbuf.at[c], gsem)
            for c in range(_NCH)
        ]
        scats = []
        for c in range(_NCH):
            gathers[c].wait()
            scats.append(
                pltpu.async_copy(rowbuf.at[c], ok_ref.at[slotbuf.at[c]], ssem))
            scats.append(
                pltpu.async_copy(rowbuf.at[c], ov_ref.at[slotbuf.at[c]], ssem))
        for s in scats:
            s.wait()

    return sc_copy, sc_scatter


def kernel(hidden_states, attention_weights, memory_keys, memory_values):
    B, S, D = hidden_states.shape
    hid = hidden_states.reshape(B * S, D)

    sc_copy, sc_scatter = _sc_kernels()
    base_k, base_v = sc_copy(memory_keys, memory_values)

    colsum3, ent_col = _stats_call(attention_weights)
    imp_row = colsum3.reshape(1, N)

    active_row, wsrc_row, wslot_row, em, ev = _rank_call(imp_row, ent_col)

    kref = jax.new_ref(base_k)
    vref = jax.new_ref(base_v)
    sc_scatter(hid, wsrc_row.reshape(32, 4, _ROWS_PER_W // 4),
               wslot_row.reshape(32, 4, _ROWS_PER_W // 4), kref, vref)

    sparse_k = kref[...]
    sparse_v = vref[...]
    return (sparse_k, sparse_v, active_row.reshape(MEM),
            em.reshape(()), ev.reshape(()))


# speedup vs baseline: 1.3548x; 1.0203x over previous
import functools

import jax
import jax.numpy as jnp
from jax import lax
from jax.experimental import pallas as pl
from jax.experimental.pallas import tpu as pltpu
from jax.experimental.pallas import tpu_sc as plsc

MEM = 2048
DIM = 1024
N = 4096
RB = 512
NBLK = 2048 // RB


def _stats_body(att_ref, colsum_ref, ent_ref, colacc_ref):
    rb = pl.program_id(1)
    a = att_ref[0]

    acc = a[0:8, :]
    for t in range(1, RB // 8):
        acc = acc + a[t * 8:(t + 1) * 8, :]
    q = acc[0:4, :] + acc[4:8, :]
    r = q[0:2, :] + q[2:4, :]
    bsum = r[0:1, :] + r[1:2, :]

    @pl.when(rb == 0)
    def _():
        colacc_ref[...] = bsum

    @pl.when(rb > 0)
    def _():
        colacc_ref[...] = colacc_ref[...] + bsum

    @pl.when(rb == NBLK - 1)
    def _():
        colsum_ref[0] = colacc_ref[...]

    z = jnp.sum(a, axis=1, keepdims=True)
    p = a / (z + 1e-10)
    entc = -jnp.sum(p * jnp.log(p + 1e-10), axis=1, keepdims=True)
    ent_ref[0] = jnp.reshape(entc, (1, RB))


_stats_call = pl.pallas_call(
    _stats_body,
    grid=(2, NBLK),
    in_specs=[pl.BlockSpec((1, RB, 2048), lambda b, rb: (b, rb, 0))],
    out_specs=[
        pl.BlockSpec((1, 1, 2048), lambda b, rb: (b, 0, 0)),
        pl.BlockSpec((1, 1, RB), lambda b, rb: (b * NBLK + rb, 0, 0)),
    ],
    out_shape=[
        jax.ShapeDtypeStruct((2, 1, 2048), jnp.float32),
        jax.ShapeDtypeStruct((2 * NBLK, 1, RB), jnp.float32),
    ],
    scratch_shapes=[pltpu.VMEM((1, 2048), jnp.float32)],
)


CH = 256


def _rank_body(imp_row_ref, ent_ref,
               active_ref, wsrc_ref, wslot_ref, em_ref, ev_ref):
    vrow = imp_row_ref[...]
    vcol = jnp.reshape(vrow, (N, 1))

    nb = N // CH
    jj = lax.broadcasted_iota(jnp.int32, (CH, CH), 0)
    ii = lax.broadcasted_iota(jnp.int32, (CH, CH), 1)
    vi_rows = [vrow[0:1, b * CH:(b + 1) * CH] for b in range(nb)]
    vj_cols = [vcol[b * CH:(b + 1) * CH] for b in range(nb)]
    acc = []
    for b in range(nb):
        win = ((vj_cols[b] > vi_rows[b])
               | ((vj_cols[b] == vi_rows[b]) & (jj < ii)))
        acc.append(jnp.sum(win.astype(jnp.float32), axis=0, keepdims=True))
    for bi in range(nb):
        for bj in range(bi + 1, nb):
            c = (vj_cols[bj] > vi_rows[bi]).astype(jnp.float32)
            acc[bi] = acc[bi] + jnp.sum(c, axis=0, keepdims=True)
            sj = jnp.sum(c, axis=1, keepdims=True)
            acc[bj] = acc[bj] + (jnp.float32(CH) - jnp.reshape(sj, (1, CH)))
    rank = jnp.concatenate(acc, axis=1)
    rank_col = jnp.reshape(rank, (N, 1))

    jcol = lax.broadcasted_iota(jnp.int32, (N, 1), 0).astype(jnp.float32)
    act_chunks = []
    for c in range(MEM // CH):
        r_row = jnp.float32(c * CH) + lax.broadcasted_iota(
            jnp.int32, (1, CH), 1).astype(jnp.float32)
        eq = (rank_col == r_row).astype(jnp.float32)
        act_chunks.append(jnp.sum(jcol * eq, axis=0, keepdims=True))
    active_ref[...] = jnp.concatenate(act_chunks, axis=1).astype(jnp.int32)

    r1 = rank[0:1, 0:MEM]
    r2 = rank[0:1, MEM:N]
    kf = jnp.float32(MEM)
    sel1 = r1 < kf
    sel2 = r2 < kf
    flag = sel1 | sel2
    s_iota = lax.broadcasted_iota(jnp.int32, (1, MEM), 1).astype(jnp.float32)
    src = jnp.where(sel1 & sel2,
                    jnp.where(r1 > r2, s_iota, s_iota + kf),
                    jnp.where(sel1, s_iota, s_iota + kf))

    flagf = flag.astype(jnp.float32)
    incl = flagf
    sh = 1
    lane = lax.broadcasted_iota(jnp.int32, (1, MEM), 1)
    while sh < MEM:
        rolled = pltpu.roll(incl, sh, 1)
        incl = incl + jnp.where(lane >= sh, rolled, 0.0)
        sh *= 2
    pos = jnp.where(flag, incl - flagf, -1.0)
    n1 = jnp.sum(flagf)

    pos_col = jnp.reshape(pos, (MEM, 1))
    src_col = jnp.reshape(src, (MEM, 1))
    s_col = lax.broadcasted_iota(jnp.int32, (MEM, 1), 0).astype(jnp.float32)
    wsrc_chunks = []
    wslot_chunks = []
    for c in range(MEM // CH):
        r_row = jnp.float32(c * CH) + lax.broadcasted_iota(
            jnp.int32, (1, CH), 1).astype(jnp.float32)
        rr = jnp.where(r_row < n1, r_row, r_row - n1)
        eq = (pos_col == rr).astype(jnp.float32)
        wsrc_chunks.append(jnp.sum(src_col * eq, axis=0, keepdims=True))
        wslot_chunks.append(jnp.sum(s_col * eq, axis=0, keepdims=True))
    wsrc_ref[...] = jnp.concatenate(wsrc_chunks, axis=1).astype(jnp.int32)
    wslot_ref[...] = jnp.concatenate(wslot_chunks, axis=1).astype(jnp.int32)

    ent = ent_ref[...]
    m = jnp.sum(ent) / jnp.float32(N)
    d = ent - m
    em_ref[...] = jnp.reshape(m, (1, 1))
    ev_ref[...] = jnp.reshape(jnp.sum(d * d) / jnp.float32(N), (1, 1))


_rank_call = pl.pallas_call(
    _rank_body,
    out_shape=[
        jax.ShapeDtypeStruct((1, MEM), jnp.int32),
        jax.ShapeDtypeStruct((1, MEM), jnp.int32),
        jax.ShapeDtypeStruct((1, MEM), jnp.int32),
        jax.ShapeDtypeStruct((1, 1), jnp.float32),
        jax.ShapeDtypeStruct((1, 1), jnp.float32),
    ],
)


_ROWS_PER_W = MEM // 32


@functools.cache
def _sc_kernels():
    mesh = plsc.VectorSubcoreMesh(core_axis_name="c", subcore_axis_name="s")

    @functools.partial(
        pl.kernel,
        mesh=mesh,
        out_type=(
            jax.ShapeDtypeStruct((MEM, DIM), jnp.float32),
            jax.ShapeDtypeStruct((MEM, DIM), jnp.float32),
        ),
        scratch_types=[pltpu.VMEM((_ROWS_PER_W, DIM), jnp.float32)],
    )
    def sc_copy(mk_hbm, mv_hbm, ok_hbm, ov_hbm, buf):
        wid = lax.axis_index("s") * 2 + lax.axis_index("c")
        base = wid * _ROWS_PER_W
        pltpu.sync_copy(mk_hbm.at[pl.ds(base, _ROWS_PER_W)], buf)
        pltpu.sync_copy(buf, ok_hbm.at[pl.ds(base, _ROWS_PER_W)])
        pltpu.sync_copy(mv_hbm.at[pl.ds(base, _ROWS_PER_W)], buf)
        pltpu.sync_copy(buf, ov_hbm.at[pl.ds(base, _ROWS_PER_W)])

    _NCH = 4
    _CW = _ROWS_PER_W // _NCH

    @functools.partial(
        pl.kernel,
        mesh=mesh,
        out_type=(),
        scratch_types=[
            pltpu.VMEM((_NCH, _CW), jnp.int32),
            pltpu.VMEM((_NCH, _CW), jnp.int32),
            pltpu.VMEM((_NCH, _CW, DIM), jnp.float32),
            pltpu.SemaphoreType.DMA,
            pltpu.SemaphoreType.DMA,
        ],
    )
    def sc_scatter(hid_hbm, wsrc_hbm, wslot_hbm, ok_ref, ov_ref,
                   srcbuf, slotbuf, rowbuf, gsem, ssem):
        wid = lax.axis_index("s") * 2 + lax.axis_index("c")
        pltpu.sync_copy(wsrc_hbm.at[wid], srcbuf)
        pltpu.sync_copy(wslot_hbm.at[wid], slotbuf)
        gathers = [
            pltpu.async_copy(hid_hbm.at[srcbuf.at[c]], rowbuf.at[c], gsem)
            for c in range(_NCH)
        ]
        scats = []
        for c in range(_NCH):
            gathers[c].wait()
            scats.append(
                pltpu.async_copy(rowbuf.at[c], ok_ref.at[slotbuf.at[c]], ssem))
            scats.append(
                pltpu.async_copy(rowbuf.at[c], ov_ref.at[slotbuf.at[c]], ssem))
        for s in scats:
            s.wait()

    return sc_copy, sc_scatter


def kernel(hidden_states, attention_weights, memory_keys, memory_values):
    B, S, D = hidden_states.shape
    hid = hidden_states.reshape(B * S, D)

    sc_copy, sc_scatter = _sc_kernels()
    base_k, base_v = sc_copy(memory_keys, memory_values)

    colsum3, ent_col = _stats_call(attention_weights)
    imp_row = colsum3.reshape(1, N)

    active_row, wsrc_row, wslot_row, em, ev = _rank_call(imp_row, ent_col)

    kref = jax.new_ref(base_k)
    vref = jax.new_ref(base_v)
    sc_scatter(hid, wsrc_row.reshape(32, 4, _ROWS_PER_W // 4),
               wslot_row.reshape(32, 4, _ROWS_PER_W // 4), kref, vref)

    sparse_k = kref[...]
    sparse_v = vref[...]
    return (sparse_k, sparse_v, active_row.reshape(MEM),
            em.reshape(()), ev.reshape(()))
